# baseline (speedup 1.0000x reference)
import functools

import jax
import jax.numpy as jnp
from jax import lax
from jax.experimental import pallas as pl
from jax.experimental.pallas import tpu as pltpu
from jax.experimental.pallas import tpu_sc as plsc

N = 320000
D = 128
G = 256

NC = 2
NS = 16
NW = NC * NS
RPW = N // NW
C = 128
CB = 512
QB = CB // C
NBIG = RPW // CB
REM = RPW - NBIG * CB
NREM = REM // C
TAIL = REM - NREM * C

BC = 16000
NBC = N // BC
W = 32


def _sc_segment_sums(x, ids):
    mesh = plsc.VectorSubcoreMesh(core_axis_name="c", subcore_axis_name="s")

    @functools.partial(
        pl.kernel,
        mesh=mesh,
        out_type=jax.ShapeDtypeStruct((NC, G, D), jnp.float32),
        scratch_types=[
            pltpu.VMEM((CB, D), jnp.float32),
            pltpu.VMEM((QB, C), jnp.int32),
            pltpu.VMEM((TAIL, D), jnp.float32),
            pltpu.VMEM((TAIL,), jnp.int32),
            pltpu.VMEM((G // NS, D), jnp.float32),
            pltpu.VMEM_SHARED((G, D), jnp.float32),
        ],
    )
    def k(x_hbm, ids_hbm, sums_out,
          rows_v, idsc_v, rowt_v, idst_v, zero_v, acc_sh):
        cid = lax.axis_index("c")
        sid = lax.axis_index("s")
        wid = sid * NC + cid

        zero16 = jnp.zeros((16,), jnp.float32)
        for r in range(G // NS):
            for q in range(D // 16):
                zero_v[r, pl.ds(q * 16, 16)] = zero16

        stripe = G // NS
        pltpu.sync_copy(zero_v, acc_sh.at[pl.ds(sid * stripe, stripe)])
        plsc.subcore_barrier()

        w_base = wid * RPW

        def body(i, carry):
            base = w_base + i * CB
            pltpu.sync_copy(x_hbm.at[pl.ds(base, CB)], rows_v)
            for q in range(QB):
                pltpu.sync_copy(ids_hbm.at[pl.ds(base + q * C, C)],
                                idsc_v.at[q])
            for q in range(QB):
                pltpu.sync_copy(rows_v.at[pl.ds(q * C, C)],
                                acc_sh.at[idsc_v.at[q]], add=True)
            return carry

        lax.fori_loop(0, NBIG, body, 0)

        rbase = w_base + NBIG * CB
        pltpu.sync_copy(x_hbm.at[pl.ds(rbase, NREM * C)],
                        rows_v.at[pl.ds(0, NREM * C)])
        for q in range(NREM):
            pltpu.sync_copy(ids_hbm.at[pl.ds(rbase + q * C, C)],
                            idsc_v.at[q])
        for q in range(NREM):
            pltpu.sync_copy(rows_v.at[pl.ds(q * C, C)],
                            acc_sh.at[idsc_v.at[q]], add=True)

        tbase = rbase + NREM * C
        pltpu.sync_copy(x_hbm.at[pl.ds(tbase, TAIL)], rowt_v)
        pltpu.sync_copy(ids_hbm.at[pl.ds(tbase, TAIL)], idst_v)
        pltpu.sync_copy(rowt_v, acc_sh.at[idst_v], add=True)

        plsc.subcore_barrier()

        @pl.when(sid == 0)
        def _emit():
            pltpu.sync_copy(acc_sh, sums_out.at[cid])

    return k(x, ids)


def _head_kernel(ids_smem, ids_ref, s_ref, w1_ref, b1_ref, w2_ref, b2_ref,
                 out_ref, cnt_ref):
    i = pl.program_id(0)

    @pl.when(i == 0)
    def _init():
        cnt_ref[...] = jnp.zeros_like(cnt_ref)

    ids = ids_ref[0, 0, :]
    first = ids_smem[0, 0, 0]
    last = ids_smem[0, 0, BC - 1]
    base = jnp.minimum((first // 8) * 8, G - W)

    @pl.when(last - base < W)
    def _local():
        seg = jax.lax.broadcasted_iota(jnp.int32, (W, BC), 0)
        onehot = (seg == (ids - base)[None, :]).astype(jnp.float32)
        c = jnp.sum(onehot, axis=1)
        cnt_ref[pl.ds(base, W), :] += jnp.broadcast_to(c[:, None], (W, 128))

    @pl.when(last - base >= W)
    def _full():
        seg = jax.lax.broadcasted_iota(jnp.int32, (G, BC), 0)
        onehot = (seg == ids[None, :]).astype(jnp.float32)
        c = jnp.sum(onehot, axis=1)
        cnt_ref[...] += jnp.broadcast_to(c[:, None], (G, 128))

    @pl.when(i == NBC - 1)
    def _finish():
        counts = jnp.maximum(cnt_ref[:, 0], 1.0)
        sums = s_ref[0, :, :] + s_ref[1, :, :]
        means = sums / counts[:, None]
        h = means @ w1_ref[...] + b1_ref[0, :][None, :]
        h = 0.5 * h * (1.0 + jax.lax.erf(h * 0.7071067811865476))
        z = h @ w2_ref[...] + b2_ref[0, :][None, :]
        out_ref[...] = jax.nn.softplus(z)


@jax.jit
def kernel(bb_feats, segment_ids, W1, b1, W2, b2):
    ids = segment_ids.astype(jnp.int32)
    sums2 = _sc_segment_sums(bb_feats, ids)

    ids3 = ids.reshape(NBC, 1, BC)
    W2p = jnp.zeros((D, 128), W2.dtype).at[:, :6].set(W2)
    b2p = jnp.zeros((1, 128), b2.dtype).at[0, :6].set(b2)
    b1p = b1.reshape(1, D)

    out = pl.pallas_call(
        _head_kernel,
        grid=(NBC,),
        in_specs=[
            pl.BlockSpec((1, 1, BC), lambda i: (i, 0, 0),
                         memory_space=pltpu.SMEM),
            pl.BlockSpec((1, 1, BC), lambda i: (i, 0, 0)),
            pl.BlockSpec((NC, G, D), lambda i: (0, 0, 0)),
            pl.BlockSpec((D, D), lambda i: (0, 0)),
            pl.BlockSpec((1, D), lambda i: (0, 0)),
            pl.BlockSpec((D, 128), lambda i: (0, 0)),
            pl.BlockSpec((1, 128), lambda i: (0, 0)),
        ],
        out_specs=pl.BlockSpec((G, 128), lambda i: (0, 0)),
        out_shape=jax.ShapeDtypeStruct((G, 128), jnp.float32),
        scratch_shapes=[
            pltpu.VMEM((G, 128), jnp.float32),
        ],
    )(ids3, ids3, sums2, W1, b1p, W2p, b2p)
    return out[:, :6]


# speedup vs baseline: 1.0329x; 1.0329x over previous
import functools

import jax
import jax.numpy as jnp
from jax import lax
from jax.experimental import pallas as pl
from jax.experimental.pallas import tpu as pltpu
from jax.experimental.pallas import tpu_sc as plsc

N = 320000
D = 128
G = 256

NC = 2
NS = 16
NW = NC * NS
RPW = N // NW
C = 128
CB = 512
QB = CB // C
NBIG = RPW // CB
REM = RPW - NBIG * CB
NREM = REM // C
TAIL = REM - NREM * C

BC = 16000
NBC = N // BC
W = 32


def _sc_segment_sums(x, ids):
    mesh = plsc.VectorSubcoreMesh(core_axis_name="c", subcore_axis_name="s")

    @functools.partial(
        pl.kernel,
        mesh=mesh,
        out_type=jax.ShapeDtypeStruct((NC, G, D), jnp.float32),
        scratch_types=[
            pltpu.VMEM((CB, D), jnp.float32),
            pltpu.VMEM((QB, C), jnp.int32),
            pltpu.VMEM((TAIL, D), jnp.float32),
            pltpu.VMEM((TAIL,), jnp.int32),
            pltpu.VMEM((G // NS, D), jnp.float32),
            pltpu.VMEM_SHARED((G, D), jnp.float32),
            pltpu.SemaphoreType.DMA,
            pltpu.SemaphoreType.DMA,
            pltpu.SemaphoreType.DMA,
            pltpu.SemaphoreType.DMA,
        ],
    )
    def k(x_hbm, ids_hbm, sums_out,
          rows_v, idsc_v, rowt_v, idst_v, zero_v, acc_sh,
          sem0, sem1, sem2, sem3):
        sems = (sem0, sem1, sem2, sem3)
        cid = lax.axis_index("c")
        sid = lax.axis_index("s")
        wid = sid * NC + cid

        zero16 = jnp.zeros((16,), jnp.float32)
        for r in range(G // NS):
            for q in range(D // 16):
                zero_v[r, pl.ds(q * 16, 16)] = zero16

        stripe = G // NS
        pltpu.sync_copy(zero_v, acc_sh.at[pl.ds(sid * stripe, stripe)])
        plsc.subcore_barrier()

        w_base = wid * RPW

        def body(i, carry):
            base = w_base + i * CB
            for q in range(QB):
                pltpu.sync_copy(ids_hbm.at[pl.ds(base + q * C, C)],
                                idsc_v.at[q])
            handles = [
                pltpu.async_copy(x_hbm.at[pl.ds(base + q * C, C)],
                                 rows_v.at[pl.ds(q * C, C)], sems[q])
                for q in range(QB)
            ]
            for q in range(QB):
                handles[q].wait()
                pltpu.sync_copy(rows_v.at[pl.ds(q * C, C)],
                                acc_sh.at[idsc_v.at[q]], add=True)
            return carry

        lax.fori_loop(0, NBIG, body, 0)

        rbase = w_base + NBIG * CB
        pltpu.sync_copy(x_hbm.at[pl.ds(rbase, NREM * C)],
                        rows_v.at[pl.ds(0, NREM * C)])
        for q in range(NREM):
            pltpu.sync_copy(ids_hbm.at[pl.ds(rbase + q * C, C)],
                            idsc_v.at[q])
        for q in range(NREM):
            pltpu.sync_copy(rows_v.at[pl.ds(q * C, C)],
                            acc_sh.at[idsc_v.at[q]], add=True)

        tbase = rbase + NREM * C
        pltpu.sync_copy(x_hbm.at[pl.ds(tbase, TAIL)], rowt_v)
        pltpu.sync_copy(ids_hbm.at[pl.ds(tbase, TAIL)], idst_v)
        pltpu.sync_copy(rowt_v, acc_sh.at[idst_v], add=True)

        plsc.subcore_barrier()

        @pl.when(sid == 0)
        def _emit():
            pltpu.sync_copy(acc_sh, sums_out.at[cid])

    return k(x, ids)


def _head_kernel(ids_smem, ids_ref, s_ref, w1_ref, b1_ref, w2_ref, b2_ref,
                 out_ref, cnt_ref):
    i = pl.program_id(0)

    @pl.when(i == 0)
    def _init():
        cnt_ref[...] = jnp.zeros_like(cnt_ref)

    ids = ids_ref[0, 0, :]
    first = ids_smem[0, 0, 0]
    last = ids_smem[0, 0, BC - 1]
    base = jnp.minimum((first // 8) * 8, G - W)

    @pl.when(last - base < W)
    def _local():
        seg = jax.lax.broadcasted_iota(jnp.int32, (W, BC), 0)
        onehot = (seg == (ids - base)[None, :]).astype(jnp.float32)
        c = jnp.sum(onehot, axis=1)
        cnt_ref[pl.ds(base, W), :] += jnp.broadcast_to(c[:, None], (W, 128))

    @pl.when(last - base >= W)
    def _full():
        seg = jax.lax.broadcasted_iota(jnp.int32, (G, BC), 0)
        onehot = (seg == ids[None, :]).astype(jnp.float32)
        c = jnp.sum(onehot, axis=1)
        cnt_ref[...] += jnp.broadcast_to(c[:, None], (G, 128))

    @pl.when(i == NBC - 1)
    def _finish():
        counts = jnp.maximum(cnt_ref[:, 0], 1.0)
        sums = s_ref[0, :, :] + s_ref[1, :, :]
        means = sums / counts[:, None]
        h = means @ w1_ref[...] + b1_ref[0, :][None, :]
        h = 0.5 * h * (1.0 + jax.lax.erf(h * 0.7071067811865476))
        z = h @ w2_ref[...] + b2_ref[0, :][None, :]
        out_ref[...] = jax.nn.softplus(z)


@jax.jit
def kernel(bb_feats, segment_ids, W1, b1, W2, b2):
    ids = segment_ids.astype(jnp.int32)
    sums2 = _sc_segment_sums(bb_feats, ids)

    ids3 = ids.reshape(NBC, 1, BC)
    W2p = jnp.zeros((D, 128), W2.dtype).at[:, :6].set(W2)
    b2p = jnp.zeros((1, 128), b2.dtype).at[0, :6].set(b2)
    b1p = b1.reshape(1, D)

    out = pl.pallas_call(
        _head_kernel,
        grid=(NBC,),
        in_specs=[
            pl.BlockSpec((1, 1, BC), lambda i: (i, 0, 0),
                         memory_space=pltpu.SMEM),
            pl.BlockSpec((1, 1, BC), lambda i: (i, 0, 0)),
            pl.BlockSpec((NC, G, D), lambda i: (0, 0, 0)),
            pl.BlockSpec((D, D), lambda i: (0, 0)),
            pl.BlockSpec((1, D), lambda i: (0, 0)),
            pl.BlockSpec((D, 128), lambda i: (0, 0)),
            pl.BlockSpec((1, 128), lambda i: (0, 0)),
        ],
        out_specs=pl.BlockSpec((G, 128), lambda i: (0, 0)),
        out_shape=jax.ShapeDtypeStruct((G, 128), jnp.float32),
        scratch_shapes=[
            pltpu.VMEM((G, 128), jnp.float32),
        ],
    )(ids3, ids3, sums2, W1, b1p, W2p, b2p)
    return out[:, :6]


# speedup vs baseline: 1.0565x; 1.0229x over previous
import functools

import jax
import jax.numpy as jnp
from jax import lax
from jax.experimental import pallas as pl
from jax.experimental.pallas import tpu as pltpu
from jax.experimental.pallas import tpu_sc as plsc

N = 320000
D = 128
G = 256

NC = 2
NS = 16
NW = NC * NS
RPW = N // NW
C = 128
NFULL = RPW // C
TAIL = RPW - NFULL * C

BC = 16000
NBC = N // BC
W = 32


def _sc_segment_sums(x, ids):
    mesh = plsc.VectorSubcoreMesh(core_axis_name="c", subcore_axis_name="s")

    @functools.partial(
        pl.kernel,
        mesh=mesh,
        out_type=jax.ShapeDtypeStruct((NC, G, D), jnp.float32),
        scratch_types=[
            pltpu.VMEM((C, D), jnp.float32),
            pltpu.VMEM((1, C), jnp.int32),
            pltpu.VMEM((TAIL, D), jnp.float32),
            pltpu.VMEM((TAIL,), jnp.int32),
            pltpu.VMEM((G // NS, D), jnp.float32),
            pltpu.VMEM_SHARED((G, D), jnp.float32),
        ],
    )
    def k(x_hbm, ids_hbm, sums_out,
          rows_v, idsc_v, rowt_v, idst_v, zero_v, acc_sh):
        cid = lax.axis_index("c")
        sid = lax.axis_index("s")
        wid = sid * NC + cid

        zero16 = jnp.zeros((16,), jnp.float32)
        for r in range(G // NS):
            for q in range(D // 16):
                zero_v[r, pl.ds(q * 16, 16)] = zero16

        stripe = G // NS
        pltpu.sync_copy(zero_v, acc_sh.at[pl.ds(sid * stripe, stripe)])
        plsc.subcore_barrier()

        w_base = wid * RPW

        def body(i, carry):
            base = w_base + i * C
            pltpu.sync_copy(x_hbm.at[pl.ds(base, C)],
                            rows_v.at[pl.ds(0, C)])
            pltpu.sync_copy(ids_hbm.at[pl.ds(base, C)], idsc_v.at[0])
            pltpu.sync_copy(rows_v.at[pl.ds(0, C)],
                            acc_sh.at[idsc_v.at[0]], add=True)
            return carry

        lax.fori_loop(0, NFULL, body, 0)

        tbase = w_base + NFULL * C
        pltpu.sync_copy(x_hbm.at[pl.ds(tbase, TAIL)], rowt_v)
        pltpu.sync_copy(ids_hbm.at[pl.ds(tbase, TAIL)], idst_v)
        pltpu.sync_copy(rowt_v, acc_sh.at[idst_v], add=True)

        plsc.subcore_barrier()

        @pl.when(sid == 0)
        def _emit():
            pltpu.sync_copy(acc_sh, sums_out.at[cid])

    return k(x, ids)


def _head_kernel(ids_smem, ids_ref, s_ref, w1_ref, b1_ref, w2_ref, b2_ref,
                 out_ref, cnt_ref):
    i = pl.program_id(0)

    @pl.when(i == 0)
    def _init():
        cnt_ref[...] = jnp.zeros_like(cnt_ref)

    ids = ids_ref[0, 0, :]
    first = ids_smem[0, 0, 0]
    last = ids_smem[0, 0, BC - 1]
    base = jnp.minimum((first // 8) * 8, G - W)

    @pl.when(last - base < W)
    def _local():
        seg = jax.lax.broadcasted_iota(jnp.int32, (W, BC), 0)
        onehot = (seg == (ids - base)[None, :]).astype(jnp.float32)
        c = jnp.sum(onehot, axis=1)
        cnt_ref[pl.ds(base, W), :] += jnp.broadcast_to(c[:, None], (W, 128))

    @pl.when(last - base >= W)
    def _full():
        seg = jax.lax.broadcasted_iota(jnp.int32, (G, BC), 0)
        onehot = (seg == ids[None, :]).astype(jnp.float32)
        c = jnp.sum(onehot, axis=1)
        cnt_ref[...] += jnp.broadcast_to(c[:, None], (G, 128))

    @pl.when(i == NBC - 1)
    def _finish():
        counts = jnp.maximum(cnt_ref[:, 0], 1.0)
        sums = s_ref[0, :, :] + s_ref[1, :, :]
        means = sums / counts[:, None]
        h = means @ w1_ref[...] + b1_ref[0, :][None, :]
        h = 0.5 * h * (1.0 + jax.lax.erf(h * 0.7071067811865476))
        z = h @ w2_ref[...] + b2_ref[0, :][None, :]
        out_ref[...] = jax.nn.softplus(z)


@jax.jit
def kernel(bb_feats, segment_ids, W1, b1, W2, b2):
    ids = segment_ids.astype(jnp.int32)
    sums2 = _sc_segment_sums(bb_feats, ids)

    ids3 = ids.reshape(NBC, 1, BC)
    W2p = jnp.zeros((D, 128), W2.dtype).at[:, :6].set(W2)
    b2p = jnp.zeros((1, 128), b2.dtype).at[0, :6].set(b2)
    b1p = b1.reshape(1, D)

    out = pl.pallas_call(
        _head_kernel,
        grid=(NBC,),
        in_specs=[
            pl.BlockSpec((1, 1, BC), lambda i: (i, 0, 0),
                         memory_space=pltpu.SMEM),
            pl.BlockSpec((1, 1, BC), lambda i: (i, 0, 0)),
            pl.BlockSpec((NC, G, D), lambda i: (0, 0, 0)),
            pl.BlockSpec((D, D), lambda i: (0, 0)),
            pl.BlockSpec((1, D), lambda i: (0, 0)),
            pl.BlockSpec((D, 128), lambda i: (0, 0)),
            pl.BlockSpec((1, 128), lambda i: (0, 0)),
        ],
        out_specs=pl.BlockSpec((G, 128), lambda i: (0, 0)),
        out_shape=jax.ShapeDtypeStruct((G, 128), jnp.float32),
        scratch_shapes=[
            pltpu.VMEM((G, 128), jnp.float32),
        ],
    )(ids3, ids3, sums2, W1, b1p, W2p, b2p)
    return out[:, :6]


# speedup vs baseline: 1.5801x; 1.4956x over previous
import functools

import jax
import jax.numpy as jnp
from jax import lax
from jax.experimental import pallas as pl
from jax.experimental.pallas import tpu as pltpu
from jax.experimental.pallas import tpu_sc as plsc

N = 320000
D = 128
G = 256

NSC = 160000
NTC = N - NSC

NC = 2
NS = 16
NW = NC * NS
RPW = NSC // NW
C = 128
NFULL = RPW // C
TAIL = RPW - NFULL * C

BX = 8000
NA = NTC // BX
BC = N // NA
W = 32


def _sc_segment_sums(x, ids):
    mesh = plsc.VectorSubcoreMesh(core_axis_name="c", subcore_axis_name="s")

    @functools.partial(
        pl.kernel,
        mesh=mesh,
        out_type=jax.ShapeDtypeStruct((NC, G, D), jnp.float32),
        scratch_types=[
            pltpu.VMEM((C, D), jnp.float32),
            pltpu.VMEM((C,), jnp.int32),
            pltpu.VMEM((TAIL, D), jnp.float32),
            pltpu.VMEM((TAIL,), jnp.int32),
            pltpu.VMEM((G // NS, D), jnp.float32),
            pltpu.VMEM_SHARED((G, D), jnp.float32),
        ],
    )
    def k(x_hbm, ids_hbm, sums_out,
          rows_v, idsc_v, rowt_v, idst_v, zero_v, acc_sh):
        cid = lax.axis_index("c")
        sid = lax.axis_index("s")
        wid = sid * NC + cid

        zero16 = jnp.zeros((16,), jnp.float32)
        for r in range(G // NS):
            for q in range(D // 16):
                zero_v[r, pl.ds(q * 16, 16)] = zero16

        stripe = G // NS
        pltpu.sync_copy(zero_v, acc_sh.at[pl.ds(sid * stripe, stripe)])
        plsc.subcore_barrier()

        w_base = (N - NSC) + wid * RPW

        def body(i, carry):
            base = w_base + i * C
            pltpu.sync_copy(x_hbm.at[pl.ds(base, C)], rows_v)
            pltpu.sync_copy(ids_hbm.at[pl.ds(base, C)], idsc_v)
            pltpu.sync_copy(rows_v, acc_sh.at[idsc_v], add=True)
            return carry

        lax.fori_loop(0, NFULL, body, 0)

        tbase = w_base + NFULL * C
        pltpu.sync_copy(x_hbm.at[pl.ds(tbase, TAIL)], rowt_v)
        pltpu.sync_copy(ids_hbm.at[pl.ds(tbase, TAIL)], idst_v)
        pltpu.sync_copy(rowt_v, acc_sh.at[idst_v], add=True)

        plsc.subcore_barrier()

        @pl.when(sid == 0)
        def _emit():
            pltpu.sync_copy(acc_sh, sums_out.at[cid])

    return k(x, ids)


def _tc_part_kernel(idsa_smem, idsa_ref, idsn_smem, idsn_ref, x_ref,
                    sums_out, cnt_out, acc_ref, cnt_ref):
    i = pl.program_id(0)

    @pl.when(i == 0)
    def _init():
        acc_ref[...] = jnp.zeros_like(acc_ref)
        cnt_ref[...] = jnp.zeros_like(cnt_ref)

    ids = idsa_ref[0, 0, :]
    x = x_ref[...]
    first = idsa_smem[0, 0, 0]
    last = idsa_smem[0, 0, BX - 1]
    base = jnp.minimum((first // 8) * 8, G - W)

    @pl.when(last - base < W)
    def _local():
        seg = jax.lax.broadcasted_iota(jnp.int32, (W, BX), 0)
        onehot = (seg == (ids - base)[None, :]).astype(jnp.float32)
        acc_ref[pl.ds(base, W), :] += jnp.dot(
            onehot, x, preferred_element_type=jnp.float32)

    @pl.when(last - base >= W)
    def _full():
        seg = jax.lax.broadcasted_iota(jnp.int32, (G, BX), 0)
        onehot = (seg == ids[None, :]).astype(jnp.float32)
        acc_ref[...] += jnp.dot(onehot, x, preferred_element_type=jnp.float32)

    idsn = idsn_ref[0, 0, :]
    first2 = idsn_smem[0, 0, 0]
    last2 = idsn_smem[0, 0, BC - 1]
    base2 = jnp.minimum((first2 // 8) * 8, G - W)

    @pl.when(last2 - base2 < W)
    def _clocal():
        seg = jax.lax.broadcasted_iota(jnp.int32, (W, BC), 0)
        onehot = (seg == (idsn - base2)[None, :]).astype(jnp.float32)
        c = jnp.sum(onehot, axis=1)
        cnt_ref[pl.ds(base2, W), :] += jnp.broadcast_to(c[:, None], (W, 128))

    @pl.when(last2 - base2 >= W)
    def _cfull():
        seg = jax.lax.broadcasted_iota(jnp.int32, (G, BC), 0)
        onehot = (seg == idsn[None, :]).astype(jnp.float32)
        c = jnp.sum(onehot, axis=1)
        cnt_ref[...] += jnp.broadcast_to(c[:, None], (G, 128))

    @pl.when(i == NA - 1)
    def _emit():
        sums_out[...] = acc_ref[...]
        cnt_out[...] = cnt_ref[...]


def _combine_kernel(s_ref, ts_ref, cnt_ref, w1_ref, b1_ref, w2_ref, b2_ref,
                    out_ref):
    counts = jnp.maximum(cnt_ref[:, 0], 1.0)
    sums = s_ref[0, :, :] + s_ref[1, :, :] + ts_ref[...]
    means = sums / counts[:, None]
    h = means @ w1_ref[...] + b1_ref[0, :][None, :]
    h = 0.5 * h * (1.0 + jax.lax.erf(h * 0.7071067811865476))
    z = h @ w2_ref[...] + b2_ref[0, :][None, :]
    out_ref[...] = jax.nn.softplus(z)


@jax.jit
def kernel(bb_feats, segment_ids, W1, b1, W2, b2):
    ids = segment_ids.astype(jnp.int32)
    sums2 = _sc_segment_sums(bb_feats, ids)

    idsa3 = ids[:NTC].reshape(NA, 1, BX)
    idsn3 = ids.reshape(NA, 1, BC)

    sums_tc, cnt = pl.pallas_call(
        _tc_part_kernel,
        grid=(NA,),
        in_specs=[
            pl.BlockSpec((1, 1, BX), lambda i: (i, 0, 0),
                         memory_space=pltpu.SMEM),
            pl.BlockSpec((1, 1, BX), lambda i: (i, 0, 0)),
            pl.BlockSpec((1, 1, BC), lambda i: (i, 0, 0),
                         memory_space=pltpu.SMEM),
            pl.BlockSpec((1, 1, BC), lambda i: (i, 0, 0)),
            pl.BlockSpec((BX, D), lambda i: (i, 0)),
        ],
        out_specs=[
            pl.BlockSpec((G, D), lambda i: (0, 0)),
            pl.BlockSpec((G, 128), lambda i: (0, 0)),
        ],
        out_shape=[
            jax.ShapeDtypeStruct((G, D), jnp.float32),
            jax.ShapeDtypeStruct((G, 128), jnp.float32),
        ],
        scratch_shapes=[
            pltpu.VMEM((G, D), jnp.float32),
            pltpu.VMEM((G, 128), jnp.float32),
        ],
    )(idsa3, idsa3, idsn3, idsn3, bb_feats[:NTC])

    W2p = jnp.zeros((D, 128), W2.dtype).at[:, :6].set(W2)
    b2p = jnp.zeros((1, 128), b2.dtype).at[0, :6].set(b2)
    b1p = b1.reshape(1, D)

    out = pl.pallas_call(
        _combine_kernel,
        out_shape=jax.ShapeDtypeStruct((G, 128), jnp.float32),
    )(sums2, sums_tc, cnt, W1, b1p, W2p, b2p)
    return out[:, :6]
